# Initial kernel scaffold; baseline (speedup 1.0000x reference)
import functools

import jax
import jax.numpy as jnp
from jax import lax
from jax.experimental import pallas as pl
from jax.experimental.pallas import tpu as pltpu
from jax.experimental.pallas import tpu_sc as plsc

N = 10000
E = 320000
D1 = 128
D2 = 16

NC = 2
NS = 16
NW = NC * NS
EPT = E // NW
NPAD = 10240
RPT = NPAD // NS

_mesh = plsc.VectorSubcoreMesh(core_axis_name="c", subcore_axis_name="s")


def _sc_agg_body(chunk, nch, d, xs_hbm, src_hbm, dst_hbm, zeros_hbm, out_hbm,
                 sidx0, sidx1, didx0, didx1, rows0, rows1, acc,
                 isem0, isem1, gsem0, gsem1):
    cid = lax.axis_index("c")
    sid = lax.axis_index("s")
    base = (cid * NS + sid) * EPT

    pltpu.sync_copy(zeros_hbm, acc.at[pl.ds(sid * RPT, RPT)])
    plsc.subcore_barrier()

    sidx = (sidx0, sidx1)
    didx = (didx0, didx1)
    rows = (rows0, rows1)
    isem = (isem0, isem1)
    gsem = (gsem0, gsem1)

    def load_idx(k):
        b = k % 2
        c1 = pltpu.async_copy(src_hbm.at[pl.ds(base + k * chunk, chunk)],
                              sidx[b], isem[b])
        c2 = pltpu.async_copy(dst_hbm.at[pl.ds(base + k * chunk, chunk)],
                              didx[b], isem[b])
        return (c1, c2)

    for c in load_idx(0):
        c.wait()
    gather = [None, None]
    gather[0] = pltpu.async_copy(xs_hbm.at[sidx[0]], rows[0], gsem[0])
    pending = load_idx(1) if nch > 1 else ()

    for k in range(nch):
        b = k % 2
        nb = (k + 1) % 2
        if k + 1 < nch:
            for c in pending:
                c.wait()
        gather[b].wait()
        if k + 1 < nch:
            gather[nb] = pltpu.async_copy(xs_hbm.at[sidx[nb]], rows[nb],
                                          gsem[nb])
            if k + 2 < nch:
                pending = load_idx(k + 2)
        pltpu.sync_copy(rows[b], acc.at[didx[b]], add=True)

    plsc.subcore_barrier()
    pltpu.sync_copy(acc.at[pl.ds(sid * RPT, RPT)],
                    out_hbm.at[cid, pl.ds(sid * RPT, RPT)])


def _sc_agg(xs, src, dst, zeros, d, chunk):
    nch = EPT // chunk
    kern = pl.kernel(
        functools.partial(_sc_agg_body, chunk, nch, d),
        out_type=jax.ShapeDtypeStruct((NC, NPAD, d), jnp.float32),
        mesh=_mesh,
        scratch_types=[
            pltpu.VMEM((chunk,), jnp.int32),
            pltpu.VMEM((chunk,), jnp.int32),
            pltpu.VMEM((chunk,), jnp.int32),
            pltpu.VMEM((chunk,), jnp.int32),
            pltpu.VMEM((chunk, d), jnp.float32),
            pltpu.VMEM((chunk, d), jnp.float32),
            pltpu.VMEM_SHARED((NPAD, d), jnp.float32),
            pltpu.SemaphoreType.DMA,
            pltpu.SemaphoreType.DMA,
            pltpu.SemaphoreType.DMA,
            pltpu.SemaphoreType.DMA,
        ],
    )
    return kern(xs, src, dst, zeros)


def _sc_deg_body(chunk, nch, dst_hbm, ones_hbm, zeros_hbm, out_hbm,
                 didx0, didx1, ones_v, acc, isem0, isem1):
    cid = lax.axis_index("c")
    sid = lax.axis_index("s")
    base = (cid * NS + sid) * EPT

    pltpu.sync_copy(zeros_hbm, acc.at[pl.ds(sid * RPT, RPT)])
    pltpu.sync_copy(ones_hbm, ones_v)
    plsc.subcore_barrier()

    didx = (didx0, didx1)
    isem = (isem0, isem1)

    def load_idx(k):
        b = k % 2
        return pltpu.async_copy(dst_hbm.at[pl.ds(base + k * chunk, chunk)],
                                didx[b], isem[b])

    pending = load_idx(0)
    for k in range(nch):
        b = k % 2
        pending.wait()
        if k + 1 < nch:
            pending = load_idx(k + 1)
        pltpu.sync_copy(ones_v, acc.at[didx[b]], add=True)

    plsc.subcore_barrier()
    pltpu.sync_copy(acc.at[pl.ds(sid * RPT, RPT)],
                    out_hbm.at[cid, pl.ds(sid * RPT, RPT)])


def _sc_deg(dst, ones, zeros, chunk):
    nch = EPT // chunk
    kern = pl.kernel(
        functools.partial(_sc_deg_body, chunk, nch),
        out_type=jax.ShapeDtypeStruct((NC, NPAD, D2), jnp.float32),
        mesh=_mesh,
        scratch_types=[
            pltpu.VMEM((chunk,), jnp.int32),
            pltpu.VMEM((chunk,), jnp.int32),
            pltpu.VMEM((chunk, D2), jnp.float32),
            pltpu.VMEM_SHARED((NPAD, D2), jnp.float32),
            pltpu.SemaphoreType.DMA,
            pltpu.SemaphoreType.DMA,
        ],
    )
    return kern(dst, ones, zeros)


def _deg_scales(dp_ref):
    dp = dp_ref[...]
    deg = 1.0 + dp[0, :, :1] + dp[1, :, :1]
    return lax.rsqrt(deg), 1.0 / deg


def _tc1_body(x_ref, w1_ref, dp_ref, xw_ref, xs_ref):
    dinv, _ = _deg_scales(dp_ref)
    xw = jnp.dot(x_ref[...], w1_ref[...], precision=lax.Precision.HIGHEST)
    xw_ref[...] = xw
    xs_ref[...] = xw * dinv


def _tc2_body(agg_ref, xw_ref, dp_ref, b1_ref, w2_ref, xw2_ref, xs2_ref):
    dinv, dinv2 = _deg_scales(dp_ref)
    a = agg_ref[...]
    h = jax.nn.relu((a[0] + a[1]) * dinv + xw_ref[...] * dinv2 + b1_ref[...])
    xw2 = jnp.dot(h, w2_ref[...], precision=lax.Precision.HIGHEST)
    xw2_ref[...] = xw2
    xs2_ref[...] = xw2 * dinv


def _tc3_body(agg_ref, xw2_ref, dp_ref, b2_ref, out_ref):
    dinv, dinv2 = _deg_scales(dp_ref)
    a = agg_ref[...]
    out_ref[...] = (a[0] + a[1]) * dinv + xw2_ref[...] * dinv2 + b2_ref[...]


_BR = 1000


def _row_spec(d):
    return pl.BlockSpec((_BR, d), lambda i: (i, 0))


def _dp_spec(d):
    return pl.BlockSpec((NC, _BR, d), lambda i: (0, i, 0))


def _full_spec(shape):
    nd = len(shape)
    return pl.BlockSpec(shape, lambda i: (0,) * nd)


def kernel(x, edge_index, W1, b1, W2, b2):
    src = edge_index[0]
    dst = edge_index[1]
    zeros1 = jnp.zeros((RPT, D1), jnp.float32)
    zeros2 = jnp.zeros((RPT, D2), jnp.float32)
    ones2 = jnp.ones((2000, D2), jnp.float32)
    b1r = b1.reshape(1, D1)
    b2r = b2.reshape(1, D2)

    grid = (N // _BR,)

    degpart = _sc_deg(dst, ones2, zeros2, chunk=2000)

    xw1, xs1 = pl.pallas_call(
        _tc1_body,
        grid=grid,
        in_specs=[_row_spec(D1), _full_spec((D1, D1)), _dp_spec(D2)],
        out_specs=[_row_spec(D1), _row_spec(D1)],
        out_shape=[jax.ShapeDtypeStruct((N, D1), jnp.float32)] * 2,
    )(x, W1, degpart)

    agg1 = _sc_agg(xs1, src, dst, zeros1, D1, chunk=400)

    xw2, xs2 = pl.pallas_call(
        _tc2_body,
        grid=grid,
        in_specs=[_dp_spec(D1), _row_spec(D1), _dp_spec(D2),
                  _full_spec((1, D1)), _full_spec((D1, D2))],
        out_specs=[_row_spec(D2), _row_spec(D2)],
        out_shape=[jax.ShapeDtypeStruct((N, D2), jnp.float32)] * 2,
    )(agg1, xw1, degpart, b1r, W2)

    agg2 = _sc_agg(xs2, src, dst, zeros2, D2, chunk=2000)

    out = pl.pallas_call(
        _tc3_body,
        grid=grid,
        in_specs=[_dp_spec(D2), _row_spec(D2), _dp_spec(D2),
                  _full_spec((1, D2))],
        out_specs=_row_spec(D2),
        out_shape=jax.ShapeDtypeStruct((N, D2), jnp.float32),
    )(agg2, xw2, degpart, b2r)

    return out


# speedup vs baseline: 21.2681x; 21.2681x over previous
import functools

import jax
import jax.numpy as jnp
from jax import lax
from jax.experimental import pallas as pl
from jax.experimental.pallas import tpu as pltpu
from jax.experimental.pallas import tpu_sc as plsc

N = 10000
E = 320000
D1 = 128
D2 = 16

NC = 2
NS = 16
NW = NC * NS
EPT = E // NW
NPAD = 10240
RPT = NPAD // NS

_mesh = plsc.VectorSubcoreMesh(core_axis_name="c", subcore_axis_name="s")


def _sc_agg_body(chunk, nch, xs_hbm, src_hbm, dst_hbm, zeros_hbm, out_hbm,
                 sidx0, sidx1, didx0, didx1, rows0, rows1, acc,
                 isem0, isem1, gsem0, gsem1):
    cid = lax.axis_index("c")
    sid = lax.axis_index("s")
    base = (cid * NS + sid) * EPT

    pltpu.sync_copy(zeros_hbm, acc.at[pl.ds(sid * RPT, RPT)])
    plsc.subcore_barrier()

    sidx = (sidx0, sidx1)
    didx = (didx0, didx1)
    rows = (rows0, rows1)
    isem = (isem0, isem1)
    gsem = (gsem0, gsem1)

    def load_idx(k):
        b = k % 2
        c1 = pltpu.async_copy(src_hbm.at[pl.ds(base + k * chunk, chunk)],
                              sidx[b], isem[b])
        c2 = pltpu.async_copy(dst_hbm.at[pl.ds(base + k * chunk, chunk)],
                              didx[b], isem[b])
        return (c1, c2)

    for c in load_idx(0):
        c.wait()
    gather = [None, None]
    gather[0] = pltpu.async_copy(xs_hbm.at[sidx[0]], rows[0], gsem[0])
    pending = load_idx(1) if nch > 1 else ()

    for k in range(nch):
        b = k % 2
        nb = (k + 1) % 2
        if k + 1 < nch:
            for c in pending:
                c.wait()
        gather[b].wait()
        if k + 1 < nch:
            gather[nb] = pltpu.async_copy(xs_hbm.at[sidx[nb]], rows[nb],
                                          gsem[nb])
            if k + 2 < nch:
                pending = load_idx(k + 2)
        pltpu.sync_copy(rows[b], acc.at[didx[b]], add=True)

    plsc.subcore_barrier()
    pltpu.sync_copy(acc.at[pl.ds(sid * RPT, RPT)],
                    out_hbm.at[cid, pl.ds(sid * RPT, RPT)])


def _sc_agg(xs, src, dst, chunk):
    nch = EPT // chunk
    zeros = jnp.zeros((RPT, D1), jnp.float32)
    kern = pl.kernel(
        functools.partial(_sc_agg_body, chunk, nch),
        out_type=jax.ShapeDtypeStruct((NC, NPAD, D1), jnp.float32),
        mesh=_mesh,
        scratch_types=[
            pltpu.VMEM((chunk,), jnp.int32),
            pltpu.VMEM((chunk,), jnp.int32),
            pltpu.VMEM((chunk,), jnp.int32),
            pltpu.VMEM((chunk,), jnp.int32),
            pltpu.VMEM((chunk, D1), jnp.float32),
            pltpu.VMEM((chunk, D1), jnp.float32),
            pltpu.VMEM_SHARED((NPAD, D1), jnp.float32),
            pltpu.SemaphoreType.DMA,
            pltpu.SemaphoreType.DMA,
            pltpu.SemaphoreType.DMA,
            pltpu.SemaphoreType.DMA,
        ],
    )
    return kern(xs, src, dst, zeros)


def _sc_deg_body(chunk, nch, dst_hbm, ones_hbm, zeros_hbm, out_hbm,
                 didx0, didx1, ones_v, acc, isem0, isem1):
    cid = lax.axis_index("c")
    sid = lax.axis_index("s")
    base = (cid * NS + sid) * EPT

    pltpu.sync_copy(zeros_hbm, acc.at[pl.ds(sid * RPT, RPT)])
    pltpu.sync_copy(ones_hbm, ones_v)
    plsc.subcore_barrier()

    didx = (didx0, didx1)
    isem = (isem0, isem1)

    def load_idx(k):
        b = k % 2
        return pltpu.async_copy(dst_hbm.at[pl.ds(base + k * chunk, chunk)],
                                didx[b], isem[b])

    pending = load_idx(0)
    for k in range(nch):
        b = k % 2
        pending.wait()
        if k + 1 < nch:
            pending = load_idx(k + 1)
        pltpu.sync_copy(ones_v, acc.at[didx[b]], add=True)

    plsc.subcore_barrier()
    pltpu.sync_copy(acc.at[pl.ds(sid * RPT, RPT)],
                    out_hbm.at[cid, pl.ds(sid * RPT, RPT)])


def _sc_deg(dst, chunk):
    nch = EPT // chunk
    ones = jnp.ones((chunk, D1), jnp.float32)
    zeros = jnp.zeros((RPT, D1), jnp.float32)
    kern = pl.kernel(
        functools.partial(_sc_deg_body, chunk, nch),
        out_type=jax.ShapeDtypeStruct((NC, NPAD, D1), jnp.float32),
        mesh=_mesh,
        scratch_types=[
            pltpu.VMEM((chunk,), jnp.int32),
            pltpu.VMEM((chunk,), jnp.int32),
            pltpu.VMEM((chunk, D1), jnp.float32),
            pltpu.VMEM_SHARED((NPAD, D1), jnp.float32),
            pltpu.SemaphoreType.DMA,
            pltpu.SemaphoreType.DMA,
        ],
    )
    return kern(dst, ones, zeros)


def _deg_scales(dp_ref):
    deg = 1.0 + dp_ref[0, :, :1] + dp_ref[1, :, :1]
    return lax.rsqrt(deg), 1.0 / deg


def _tc1_body(x_ref, w1_ref, dp_ref, xw_ref, xs_ref):
    dinv, _ = _deg_scales(dp_ref)
    xw = jnp.dot(x_ref[...], w1_ref[...], precision=lax.Precision.HIGHEST)
    xw_ref[...] = xw
    xs_ref[...] = xw * dinv


def _tc2_body(agg_ref, xw_ref, dp_ref, b1_ref, h_ref, hs_ref):
    dinv, dinv2 = _deg_scales(dp_ref)
    h = jax.nn.relu((agg_ref[0] + agg_ref[1]) * dinv
                    + xw_ref[...] * dinv2 + b1_ref[...])
    h_ref[...] = h
    hs_ref[...] = h * dinv


def _tc3_body(agg_ref, h_ref, dp_ref, b2_ref, w2_ref, out_ref):
    dinv, dinv2 = _deg_scales(dp_ref)
    ah = (agg_ref[0] + agg_ref[1]) * dinv + h_ref[...] * dinv2
    out_ref[...] = jnp.dot(ah, w2_ref[...],
                           precision=lax.Precision.HIGHEST) + b2_ref[...]


_BR = 1000


def _row_spec(d):
    return pl.BlockSpec((_BR, d), lambda i: (i, 0))


def _part_spec():
    return pl.BlockSpec((NC, _BR, D1), lambda i: (0, i, 0))


def _full_spec(shape):
    nd = len(shape)
    return pl.BlockSpec(shape, lambda i: (0,) * nd)


def kernel(x, edge_index, W1, b1, W2, b2):
    src = edge_index[0]
    dst = edge_index[1]
    b1r = b1.reshape(1, D1)
    b2r = b2.reshape(1, D2)

    grid = (N // _BR,)

    degpart = _sc_deg(dst, chunk=200)

    xw1, xs1 = pl.pallas_call(
        _tc1_body,
        grid=grid,
        in_specs=[_row_spec(D1), _full_spec((D1, D1)), _part_spec()],
        out_specs=[_row_spec(D1), _row_spec(D1)],
        out_shape=[jax.ShapeDtypeStruct((N, D1), jnp.float32)] * 2,
    )(x, W1, degpart)

    agg1 = _sc_agg(xs1, src, dst, chunk=80)

    h, hs = pl.pallas_call(
        _tc2_body,
        grid=grid,
        in_specs=[_part_spec(), _row_spec(D1), _part_spec(),
                  _full_spec((1, D1))],
        out_specs=[_row_spec(D1), _row_spec(D1)],
        out_shape=[jax.ShapeDtypeStruct((N, D1), jnp.float32)] * 2,
    )(agg1, xw1, degpart, b1r)

    agg2 = _sc_agg(hs, src, dst, chunk=80)

    out = pl.pallas_call(
        _tc3_body,
        grid=grid,
        in_specs=[_part_spec(), _row_spec(D1), _part_spec(),
                  _full_spec((1, D2)), _full_spec((D1, D2))],
        out_specs=_row_spec(D2),
        out_shape=jax.ShapeDtypeStruct((N, D2), jnp.float32),
    )(agg2, h, degpart, b2r, W2)

    return out


# speedup vs baseline: 23.8649x; 1.1221x over previous
import dataclasses
import functools

import jax
import jax.numpy as jnp
from jax import lax
from jax.experimental import pallas as pl
from jax.experimental.pallas import tpu as pltpu
from jax.experimental.pallas import tpu_sc as plsc

N = 10000
E = 320000
D1 = 128
D2 = 16

NC = 2
NS = 16
NW = NC * NS
EPT = E // NW
NPAD = 10240
RPT = NPAD // NS

_mesh = plsc.VectorSubcoreMesh(core_axis_name="c", subcore_axis_name="s")


def _sc_agg_body(chunk, nch, xs_hbm, src_hbm, dst_hbm, zeros_hbm, out_hbm,
                 sidx0, sidx1, didx0, didx1, rows0, rows1, acc,
                 isem0, isem1, gsem0, gsem1):
    cid = lax.axis_index("c")
    sid = lax.axis_index("s")
    base = (cid * NS + sid) * EPT

    pltpu.sync_copy(zeros_hbm, acc.at[pl.ds(sid * RPT, RPT)])
    plsc.subcore_barrier()

    sidx = (sidx0, sidx1)
    didx = (didx0, didx1)
    rows = (rows0, rows1)
    isem = (isem0, isem1)
    gsem = (gsem0, gsem1)

    def load_idx(k):
        b = k % 2
        c1 = pltpu.async_copy(src_hbm.at[pl.ds(base + k * chunk, chunk)],
                              sidx[b], isem[b])
        c2 = pltpu.async_copy(dst_hbm.at[pl.ds(base + k * chunk, chunk)],
                              didx[b], isem[b])
        return (c1, c2)

    for c in load_idx(0):
        c.wait()
    gather = [None, None]
    gather[0] = pltpu.async_copy(xs_hbm.at[sidx[0]], rows[0], gsem[0])
    pending = load_idx(1) if nch > 1 else ()

    for k in range(nch):
        b = k % 2
        nb = (k + 1) % 2
        if k + 1 < nch:
            for c in pending:
                c.wait()
        gather[b].wait()
        if k + 1 < nch:
            gather[nb] = pltpu.async_copy(xs_hbm.at[sidx[nb]], rows[nb],
                                          gsem[nb])
            if k + 2 < nch:
                pending = load_idx(k + 2)
        pltpu.sync_copy(rows[b], acc.at[didx[b]], add=True)

    plsc.subcore_barrier()
    pltpu.sync_copy(acc.at[pl.ds(sid * RPT, RPT)],
                    out_hbm.at[cid, pl.ds(sid * RPT, RPT)])


def _sc_agg(xs, src, dst, chunk):
    nch = EPT // chunk
    zeros = jnp.zeros((RPT, D1), jnp.float32)
    kern = pl.kernel(
        functools.partial(_sc_agg_body, chunk, nch),
        out_type=jax.ShapeDtypeStruct((NC, NPAD, D1), jnp.float32),
        mesh=_mesh,
        scratch_types=[
            pltpu.VMEM((chunk,), jnp.int32),
            pltpu.VMEM((chunk,), jnp.int32),
            pltpu.VMEM((chunk,), jnp.int32),
            pltpu.VMEM((chunk,), jnp.int32),
            pltpu.VMEM((chunk, D1), jnp.float32),
            pltpu.VMEM((chunk, D1), jnp.float32),
            pltpu.VMEM_SHARED((NPAD, D1), jnp.float32),
            pltpu.SemaphoreType.DMA,
            pltpu.SemaphoreType.DMA,
            pltpu.SemaphoreType.DMA,
            pltpu.SemaphoreType.DMA,
        ],
    )
    return kern(xs, src, dst, zeros)


def _sc_hist_body(chunk, nch, dst_hbm, zeros_hbm, out_hbm,
                  didx0, didx1, hist, isem0, isem1):
    cid = lax.axis_index("c")
    sid = lax.axis_index("s")
    base = (cid * NS + sid) * EPT

    pltpu.sync_copy(zeros_hbm, hist)

    didx = (didx0, didx1)
    isem = (isem0, isem1)

    def load_idx(k):
        b = k % 2
        return pltpu.async_copy(dst_hbm.at[pl.ds(base + k * chunk, chunk)],
                                didx[b], isem[b])

    pending = load_idx(0)
    for k in range(nch):
        b = k % 2
        pending.wait()
        if k + 1 < nch:
            pending = load_idx(k + 1)

        @pl.loop(0, chunk, step=16)
        def _(j):
            v = didx[b][pl.ds(j, 16)]
            cnt, last = plsc.scan_count(v)
            plsc.addupdate_scatter(hist, [v], cnt, mask=last)

    pltpu.sync_copy(hist, out_hbm.at[cid * NS + sid])


def _sc_hist(dst, chunk):
    nch = EPT // chunk
    zeros = jnp.zeros((NPAD,), jnp.int32)
    cp = pltpu.CompilerParams()
    if "needs_layout_passes" in pltpu.CompilerParams.__dataclass_fields__:
        cp = dataclasses.replace(cp, needs_layout_passes=False)
    kern = pl.kernel(
        functools.partial(_sc_hist_body, chunk, nch),
        out_type=jax.ShapeDtypeStruct((NW, NPAD), jnp.int32),
        mesh=_mesh,
        compiler_params=cp,
        scratch_types=[
            pltpu.VMEM((chunk,), jnp.int32),
            pltpu.VMEM((chunk,), jnp.int32),
            pltpu.VMEM((NPAD,), jnp.int32),
            pltpu.SemaphoreType.DMA,
            pltpu.SemaphoreType.DMA,
        ],
    )
    return kern(dst, zeros)


_BD = 2048


def _tc0_body(hist_ref, dinv_ref, dinv2_ref):
    counts = jnp.sum(hist_ref[...].astype(jnp.float32), axis=0,
                     keepdims=True)
    deg = 1.0 + jnp.transpose(counts)
    dinv_ref[...] = lax.rsqrt(deg)
    dinv2_ref[...] = 1.0 / deg


def _tc1_body(x_ref, w1_ref, dinv_ref, xw_ref, xs_ref):
    xw = jnp.dot(x_ref[...], w1_ref[...], precision=lax.Precision.HIGHEST)
    xw_ref[...] = xw
    xs_ref[...] = xw * dinv_ref[...]


def _tc2_body(agg_ref, xw_ref, dinv_ref, dinv2_ref, b1_ref, h_ref, hs_ref):
    dinv = dinv_ref[...]
    h = jax.nn.relu((agg_ref[0] + agg_ref[1]) * dinv
                    + xw_ref[...] * dinv2_ref[...] + b1_ref[...])
    h_ref[...] = h
    hs_ref[...] = h * dinv


def _tc3_body(agg_ref, h_ref, dinv_ref, dinv2_ref, b2_ref, w2_ref, out_ref):
    ah = (agg_ref[0] + agg_ref[1]) * dinv_ref[...] \
        + h_ref[...] * dinv2_ref[...]
    out_ref[...] = jnp.dot(ah, w2_ref[...],
                           precision=lax.Precision.HIGHEST) + b2_ref[...]


_BR = 1000


def _row_spec(d):
    return pl.BlockSpec((_BR, d), lambda i: (i, 0))


def _part_spec():
    return pl.BlockSpec((NC, _BR, D1), lambda i: (0, i, 0))


def _full_spec(shape):
    nd = len(shape)
    return pl.BlockSpec(shape, lambda i: (0,) * nd)


def kernel(x, edge_index, W1, b1, W2, b2):
    src = edge_index[0]
    dst = edge_index[1]
    b1r = b1.reshape(1, D1)
    b2r = b2.reshape(1, D2)

    grid = (N // _BR,)
    dspec = _row_spec(1)

    hist = _sc_hist(dst, chunk=2000)

    dinv, dinv2 = pl.pallas_call(
        _tc0_body,
        grid=(NPAD // _BD,),
        in_specs=[pl.BlockSpec((NW, _BD), lambda i: (0, i))],
        out_specs=[pl.BlockSpec((_BD, 1), lambda i: (i, 0))] * 2,
        out_shape=[jax.ShapeDtypeStruct((NPAD, 1), jnp.float32)] * 2,
    )(hist)

    xw1, xs1 = pl.pallas_call(
        _tc1_body,
        grid=grid,
        in_specs=[_row_spec(D1), _full_spec((D1, D1)), dspec],
        out_specs=[_row_spec(D1), _row_spec(D1)],
        out_shape=[jax.ShapeDtypeStruct((N, D1), jnp.float32)] * 2,
    )(x, W1, dinv)

    agg1 = _sc_agg(xs1, src, dst, chunk=80)

    h, hs = pl.pallas_call(
        _tc2_body,
        grid=grid,
        in_specs=[_part_spec(), _row_spec(D1), dspec, dspec,
                  _full_spec((1, D1))],
        out_specs=[_row_spec(D1), _row_spec(D1)],
        out_shape=[jax.ShapeDtypeStruct((N, D1), jnp.float32)] * 2,
    )(agg1, xw1, dinv, dinv2, b1r)

    agg2 = _sc_agg(hs, src, dst, chunk=80)

    out = pl.pallas_call(
        _tc3_body,
        grid=grid,
        in_specs=[_part_spec(), _row_spec(D1), dspec, dspec,
                  _full_spec((1, D2)), _full_spec((D1, D2))],
        out_specs=_row_spec(D2),
        out_shape=jax.ShapeDtypeStruct((N, D2), jnp.float32),
    )(agg2, h, dinv, dinv2, b2r, W2)

    return out


# speedup vs baseline: 32.3813x; 1.3569x over previous
import dataclasses
import functools

import jax
import jax.numpy as jnp
from jax import lax
from jax.experimental import pallas as pl
from jax.experimental.pallas import tpu as pltpu
from jax.experimental.pallas import tpu_sc as plsc

N = 10000
E = 320000
D1 = 128
D2 = 16

NC = 2
NS = 16
NW = NC * NS
EPT = E // NW
NPAD = 10240
RPT = NPAD // NS

_mesh = plsc.VectorSubcoreMesh(core_axis_name="c", subcore_axis_name="s")


_NROW = 3
_NIDX = 4


def _sc_agg_body(chunk, nch, xs_hbm, src_hbm, dst_hbm, zeros_hbm, out_hbm,
                 *refs):
    sidx = refs[0:_NIDX]
    didx = refs[_NIDX:2 * _NIDX]
    rows = refs[2 * _NIDX:2 * _NIDX + _NROW]
    acc = refs[2 * _NIDX + _NROW]
    isem = refs[2 * _NIDX + _NROW + 1:3 * _NIDX + _NROW + 1]
    gsem = refs[3 * _NIDX + _NROW + 1:3 * _NIDX + 2 * _NROW + 1]
    ssem = refs[3 * _NIDX + 2 * _NROW + 1:3 * _NIDX + 3 * _NROW + 1]

    cid = lax.axis_index("c")
    sid = lax.axis_index("s")
    base = (cid * NS + sid) * EPT

    pltpu.sync_copy(zeros_hbm, acc.at[pl.ds(sid * RPT, RPT)])
    plsc.subcore_barrier()

    def load_idx(k):
        b = k % _NIDX
        c1 = pltpu.async_copy(src_hbm.at[pl.ds(base + k * chunk, chunk)],
                              sidx[b], isem[b])
        c2 = pltpu.async_copy(dst_hbm.at[pl.ds(base + k * chunk, chunk)],
                              didx[b], isem[b])
        return (c1, c2)

    def gather(k):
        return pltpu.async_copy(xs_hbm.at[sidx[k % _NIDX]],
                                rows[k % _NROW], gsem[k % _NROW])

    def scatter(k):
        return pltpu.async_copy(rows[k % _NROW], acc.at[didx[k % _NIDX]],
                                ssem[k % _NROW], add=True)

    idx_pending = [None] * _NIDX
    for k in range(min(3, nch)):
        idx_pending[k % _NIDX] = load_idx(k)
    g = [None] * nch
    s = [None] * nch
    for k in range(min(2, nch)):
        for c in idx_pending[k % _NIDX]:
            c.wait()
        g[k] = gather(k)

    for k in range(nch):
        g[k].wait()
        if k - 1 >= 0:
            s[k - 1].wait()
        if k + 3 < nch:
            idx_pending[(k + 3) % _NIDX] = load_idx(k + 3)
        if k + 2 < nch:
            for c in idx_pending[(k + 2) % _NIDX]:
                c.wait()
            g[k + 2] = gather(k + 2)
        s[k] = scatter(k)

    s[nch - 1].wait()

    plsc.subcore_barrier()
    pltpu.sync_copy(acc.at[pl.ds(sid * RPT, RPT)],
                    out_hbm.at[cid, pl.ds(sid * RPT, RPT)])


def _sc_agg(xs, src, dst, chunk):
    nch = EPT // chunk
    zeros = jnp.zeros((RPT, D1), jnp.float32)
    scratch = (
        [pltpu.VMEM((chunk,), jnp.int32)] * _NIDX
        + [pltpu.VMEM((chunk,), jnp.int32)] * _NIDX
        + [pltpu.VMEM((chunk, D1), jnp.float32)] * _NROW
        + [pltpu.VMEM_SHARED((NPAD, D1), jnp.float32)]
        + [pltpu.SemaphoreType.DMA] * _NIDX
        + [pltpu.SemaphoreType.DMA] * (2 * _NROW)
    )
    kern = pl.kernel(
        functools.partial(_sc_agg_body, chunk, nch),
        out_type=jax.ShapeDtypeStruct((NC, NPAD, D1), jnp.float32),
        mesh=_mesh,
        scratch_types=scratch,
    )
    return kern(xs, src, dst, zeros)


def _sc_hist_body(chunk, nch, dst_hbm, zeros_hbm, out_hbm,
                  didx0, didx1, hist, isem0, isem1):
    cid = lax.axis_index("c")
    sid = lax.axis_index("s")
    base = (cid * NS + sid) * EPT

    pltpu.sync_copy(zeros_hbm, hist)

    didx = (didx0, didx1)
    isem = (isem0, isem1)

    def load_idx(k):
        b = k % 2
        return pltpu.async_copy(dst_hbm.at[pl.ds(base + k * chunk, chunk)],
                                didx[b], isem[b])

    pending = load_idx(0)
    for k in range(nch):
        b = k % 2
        pending.wait()
        if k + 1 < nch:
            pending = load_idx(k + 1)

        @pl.loop(0, chunk, step=16)
        def _(j):
            v = didx[b][pl.ds(j, 16)]
            cnt, last = plsc.scan_count(v)
            plsc.addupdate_scatter(hist, [v], cnt, mask=last)

    pltpu.sync_copy(hist, out_hbm.at[cid * NS + sid])


def _sc_hist(dst, chunk):
    nch = EPT // chunk
    zeros = jnp.zeros((NPAD,), jnp.int32)
    cp = pltpu.CompilerParams()
    if "needs_layout_passes" in pltpu.CompilerParams.__dataclass_fields__:
        cp = dataclasses.replace(cp, needs_layout_passes=False)
    kern = pl.kernel(
        functools.partial(_sc_hist_body, chunk, nch),
        out_type=jax.ShapeDtypeStruct((NW, NPAD), jnp.int32),
        mesh=_mesh,
        compiler_params=cp,
        scratch_types=[
            pltpu.VMEM((chunk,), jnp.int32),
            pltpu.VMEM((chunk,), jnp.int32),
            pltpu.VMEM((NPAD,), jnp.int32),
            pltpu.SemaphoreType.DMA,
            pltpu.SemaphoreType.DMA,
        ],
    )
    return kern(dst, zeros)


_BD = 2048


def _tc0_body(hist_ref, dinv_ref, dinv2_ref):
    counts = jnp.sum(hist_ref[...].astype(jnp.float32), axis=0,
                     keepdims=True)
    deg = 1.0 + jnp.transpose(counts)
    dinv_ref[...] = lax.rsqrt(deg)
    dinv2_ref[...] = 1.0 / deg


def _tc1_body(x_ref, w1_ref, dinv_ref, xw_ref, xs_ref):
    xw = jnp.dot(x_ref[...], w1_ref[...], precision=lax.Precision.HIGHEST)
    xw_ref[...] = xw
    xs_ref[...] = xw * dinv_ref[...]


def _tc2_body(agg_ref, xw_ref, dinv_ref, dinv2_ref, b1_ref, h_ref, hs_ref):
    dinv = dinv_ref[...]
    h = jax.nn.relu((agg_ref[0] + agg_ref[1]) * dinv
                    + xw_ref[...] * dinv2_ref[...] + b1_ref[...])
    h_ref[...] = h
    hs_ref[...] = h * dinv


def _tc3_body(agg_ref, h_ref, dinv_ref, dinv2_ref, b2_ref, w2_ref, out_ref):
    ah = (agg_ref[0] + agg_ref[1]) * dinv_ref[...] \
        + h_ref[...] * dinv2_ref[...]
    out_ref[...] = jnp.dot(ah, w2_ref[...],
                           precision=lax.Precision.HIGHEST) + b2_ref[...]


_BR = 1000


def _row_spec(d):
    return pl.BlockSpec((_BR, d), lambda i: (i, 0))


def _part_spec():
    return pl.BlockSpec((NC, _BR, D1), lambda i: (0, i, 0))


def _full_spec(shape):
    nd = len(shape)
    return pl.BlockSpec(shape, lambda i: (0,) * nd)


def kernel(x, edge_index, W1, b1, W2, b2):
    src = edge_index[0]
    dst = edge_index[1]
    b1r = b1.reshape(1, D1)
    b2r = b2.reshape(1, D2)

    grid = (N // _BR,)
    dspec = _row_spec(1)

    hist = _sc_hist(dst, chunk=2000)

    dinv, dinv2 = pl.pallas_call(
        _tc0_body,
        grid=(NPAD // _BD,),
        in_specs=[pl.BlockSpec((NW, _BD), lambda i: (0, i))],
        out_specs=[pl.BlockSpec((_BD, 1), lambda i: (i, 0))] * 2,
        out_shape=[jax.ShapeDtypeStruct((NPAD, 1), jnp.float32)] * 2,
    )(hist)

    xw1, xs1 = pl.pallas_call(
        _tc1_body,
        grid=grid,
        in_specs=[_row_spec(D1), _full_spec((D1, D1)), dspec],
        out_specs=[_row_spec(D1), _row_spec(D1)],
        out_shape=[jax.ShapeDtypeStruct((N, D1), jnp.float32)] * 2,
    )(x, W1, dinv)

    agg1 = _sc_agg(xs1, src, dst, chunk=80)

    h, hs = pl.pallas_call(
        _tc2_body,
        grid=grid,
        in_specs=[_part_spec(), _row_spec(D1), dspec, dspec,
                  _full_spec((1, D1))],
        out_specs=[_row_spec(D1), _row_spec(D1)],
        out_shape=[jax.ShapeDtypeStruct((N, D1), jnp.float32)] * 2,
    )(agg1, xw1, dinv, dinv2, b1r)

    agg2 = _sc_agg(hs, src, dst, chunk=80)

    out = pl.pallas_call(
        _tc3_body,
        grid=grid,
        in_specs=[_part_spec(), _row_spec(D1), dspec, dspec,
                  _full_spec((1, D2)), _full_spec((D1, D2))],
        out_specs=_row_spec(D2),
        out_shape=jax.ShapeDtypeStruct((N, D2), jnp.float32),
    )(agg2, h, dinv, dinv2, b2r, W2)

    return out


# speedup vs baseline: 33.7641x; 1.0427x over previous
import dataclasses
import functools

import jax
import jax.numpy as jnp
from jax import lax
from jax.experimental import pallas as pl
from jax.experimental.pallas import tpu as pltpu
from jax.experimental.pallas import tpu_sc as plsc

N = 10000
E = 320000
D1 = 128
D2 = 16

NC = 2
NS = 16
NW = NC * NS
EPT = E // NW
NPAD = 10240
RPT = NPAD // NS

_mesh = plsc.VectorSubcoreMesh(core_axis_name="c", subcore_axis_name="s")


_NROW = 4
_NIDX = 5


def _sc_agg_body(chunk, nch, xs_hbm, src_hbm, dst_hbm, zeros_hbm, out_hbm,
                 *refs):
    sidx = refs[0:_NIDX]
    didx = refs[_NIDX:2 * _NIDX]
    rows = refs[2 * _NIDX:2 * _NIDX + _NROW]
    acc = refs[2 * _NIDX + _NROW]
    isem = refs[2 * _NIDX + _NROW + 1:3 * _NIDX + _NROW + 1]
    gsem = refs[3 * _NIDX + _NROW + 1:3 * _NIDX + 2 * _NROW + 1]
    ssem = refs[3 * _NIDX + 2 * _NROW + 1:3 * _NIDX + 3 * _NROW + 1]

    cid = lax.axis_index("c")
    sid = lax.axis_index("s")
    base = (cid * NS + sid) * EPT

    pltpu.sync_copy(zeros_hbm, acc.at[pl.ds(sid * RPT, RPT)])
    plsc.subcore_barrier()

    def load_idx(k):
        b = k % _NIDX
        c1 = pltpu.async_copy(src_hbm.at[pl.ds(base + k * chunk, chunk)],
                              sidx[b], isem[b])
        c2 = pltpu.async_copy(dst_hbm.at[pl.ds(base + k * chunk, chunk)],
                              didx[b], isem[b])
        return (c1, c2)

    def gather(k):
        return pltpu.async_copy(xs_hbm.at[sidx[k % _NIDX]],
                                rows[k % _NROW], gsem[k % _NROW])

    def scatter(k):
        return pltpu.async_copy(rows[k % _NROW], acc.at[didx[k % _NIDX]],
                                ssem[k % _NROW], add=True)

    ng = _NROW - 1
    idx_pending = [None] * _NIDX
    for k in range(min(ng + 1, nch)):
        idx_pending[k % _NIDX] = load_idx(k)
    g = [None] * nch
    s = [None] * nch
    for k in range(min(ng, nch)):
        for c in idx_pending[k % _NIDX]:
            c.wait()
        g[k] = gather(k)

    for k in range(nch):
        g[k].wait()
        if k - 1 >= 0:
            s[k - 1].wait()
        if k + ng + 1 < nch:
            idx_pending[(k + ng + 1) % _NIDX] = load_idx(k + ng + 1)
        if k + ng < nch:
            for c in idx_pending[(k + ng) % _NIDX]:
                c.wait()
            g[k + ng] = gather(k + ng)
        s[k] = scatter(k)

    s[nch - 1].wait()

    plsc.subcore_barrier()
    pltpu.sync_copy(acc.at[pl.ds(sid * RPT, RPT)],
                    out_hbm.at[cid, pl.ds(sid * RPT, RPT)])


def _sc_agg(xs, src, dst, chunk):
    nch = EPT // chunk
    zeros = jnp.zeros((RPT, D1), jnp.float32)
    scratch = (
        [pltpu.VMEM((chunk,), jnp.int32)] * _NIDX
        + [pltpu.VMEM((chunk,), jnp.int32)] * _NIDX
        + [pltpu.VMEM((chunk, D1), jnp.float32)] * _NROW
        + [pltpu.VMEM_SHARED((NPAD, D1), jnp.float32)]
        + [pltpu.SemaphoreType.DMA] * _NIDX
        + [pltpu.SemaphoreType.DMA] * (2 * _NROW)
    )
    kern = pl.kernel(
        functools.partial(_sc_agg_body, chunk, nch),
        out_type=jax.ShapeDtypeStruct((NC, NPAD, D1), jnp.float32),
        mesh=_mesh,
        scratch_types=scratch,
    )
    return kern(xs, src, dst, zeros)


def _sc_hist_body(chunk, nch, dst_hbm, zeros_hbm, out_hbm,
                  didx0, didx1, hist, isem0, isem1):
    cid = lax.axis_index("c")
    sid = lax.axis_index("s")
    base = (cid * NS + sid) * EPT

    pltpu.sync_copy(zeros_hbm, hist)

    didx = (didx0, didx1)
    isem = (isem0, isem1)

    def load_idx(k):
        b = k % 2
        return pltpu.async_copy(dst_hbm.at[pl.ds(base + k * chunk, chunk)],
                                didx[b], isem[b])

    pending = load_idx(0)
    for k in range(nch):
        b = k % 2
        pending.wait()
        if k + 1 < nch:
            pending = load_idx(k + 1)

        @pl.loop(0, chunk, step=16)
        def _(j):
            v = didx[b][pl.ds(j, 16)]
            cnt, last = plsc.scan_count(v)
            plsc.addupdate_scatter(hist, [v], cnt, mask=last)

    pltpu.sync_copy(hist, out_hbm.at[cid * NS + sid])


def _sc_hist(dst, chunk):
    nch = EPT // chunk
    zeros = jnp.zeros((NPAD,), jnp.int32)
    cp = pltpu.CompilerParams()
    if "needs_layout_passes" in pltpu.CompilerParams.__dataclass_fields__:
        cp = dataclasses.replace(cp, needs_layout_passes=False)
    kern = pl.kernel(
        functools.partial(_sc_hist_body, chunk, nch),
        out_type=jax.ShapeDtypeStruct((NW, NPAD), jnp.int32),
        mesh=_mesh,
        compiler_params=cp,
        scratch_types=[
            pltpu.VMEM((chunk,), jnp.int32),
            pltpu.VMEM((chunk,), jnp.int32),
            pltpu.VMEM((NPAD,), jnp.int32),
            pltpu.SemaphoreType.DMA,
            pltpu.SemaphoreType.DMA,
        ],
    )
    return kern(dst, zeros)


_BD = 2048


def _tc0_body(hist_ref, dinv_ref, dinv2_ref):
    counts = jnp.sum(hist_ref[...].astype(jnp.float32), axis=0,
                     keepdims=True)
    deg = 1.0 + jnp.transpose(counts)
    dinv_ref[...] = lax.rsqrt(deg)
    dinv2_ref[...] = 1.0 / deg


def _tc1_body(x_ref, w1_ref, dinv_ref, xw_ref, xs_ref):
    xw = jnp.dot(x_ref[...], w1_ref[...], precision=lax.Precision.HIGHEST)
    xw_ref[...] = xw
    xs_ref[...] = xw * dinv_ref[...]


def _tc2_body(agg_ref, xw_ref, dinv_ref, dinv2_ref, b1_ref, h_ref, hs_ref):
    dinv = dinv_ref[...]
    h = jax.nn.relu((agg_ref[0] + agg_ref[1]) * dinv
                    + xw_ref[...] * dinv2_ref[...] + b1_ref[...])
    h_ref[...] = h
    hs_ref[...] = h * dinv


def _tc3_body(agg_ref, h_ref, dinv_ref, dinv2_ref, b2_ref, w2_ref, out_ref):
    ah = (agg_ref[0] + agg_ref[1]) * dinv_ref[...] \
        + h_ref[...] * dinv2_ref[...]
    out_ref[...] = jnp.dot(ah, w2_ref[...],
                           precision=lax.Precision.HIGHEST) + b2_ref[...]


_BR = 1000


def _row_spec(d):
    return pl.BlockSpec((_BR, d), lambda i: (i, 0))


def _part_spec():
    return pl.BlockSpec((NC, _BR, D1), lambda i: (0, i, 0))


def _full_spec(shape):
    nd = len(shape)
    return pl.BlockSpec(shape, lambda i: (0,) * nd)


def kernel(x, edge_index, W1, b1, W2, b2):
    src = edge_index[0]
    dst = edge_index[1]
    b1r = b1.reshape(1, D1)
    b2r = b2.reshape(1, D2)

    grid = (N // _BR,)
    dspec = _row_spec(1)

    hist = _sc_hist(dst, chunk=2000)

    dinv, dinv2 = pl.pallas_call(
        _tc0_body,
        grid=(NPAD // _BD,),
        in_specs=[pl.BlockSpec((NW, _BD), lambda i: (0, i))],
        out_specs=[pl.BlockSpec((_BD, 1), lambda i: (i, 0))] * 2,
        out_shape=[jax.ShapeDtypeStruct((NPAD, 1), jnp.float32)] * 2,
    )(hist)

    xw1, xs1 = pl.pallas_call(
        _tc1_body,
        grid=grid,
        in_specs=[_row_spec(D1), _full_spec((D1, D1)), dspec],
        out_specs=[_row_spec(D1), _row_spec(D1)],
        out_shape=[jax.ShapeDtypeStruct((N, D1), jnp.float32)] * 2,
    )(x, W1, dinv)

    agg1 = _sc_agg(xs1, src, dst, chunk=80)

    h, hs = pl.pallas_call(
        _tc2_body,
        grid=grid,
        in_specs=[_part_spec(), _row_spec(D1), dspec, dspec,
                  _full_spec((1, D1))],
        out_specs=[_row_spec(D1), _row_spec(D1)],
        out_shape=[jax.ShapeDtypeStruct((N, D1), jnp.float32)] * 2,
    )(agg1, xw1, dinv, dinv2, b1r)

    agg2 = _sc_agg(hs, src, dst, chunk=80)

    out = pl.pallas_call(
        _tc3_body,
        grid=grid,
        in_specs=[_part_spec(), _row_spec(D1), dspec, dspec,
                  _full_spec((1, D2)), _full_spec((D1, D2))],
        out_specs=_row_spec(D2),
        out_shape=jax.ShapeDtypeStruct((N, D2), jnp.float32),
    )(agg2, h, dinv, dinv2, b2r, W2)

    return out


# speedup vs baseline: 34.7471x; 1.0291x over previous
import dataclasses
import functools

import jax
import jax.numpy as jnp
from jax import lax
from jax.experimental import pallas as pl
from jax.experimental.pallas import tpu as pltpu
from jax.experimental.pallas import tpu_sc as plsc

N = 10000
E = 320000
D1 = 128
D2 = 16

NC = 2
NS = 16
NW = NC * NS
EPT = E // NW
NPAD = 10240
RPT = NPAD // NS

_mesh = plsc.VectorSubcoreMesh(core_axis_name="c", subcore_axis_name="s")


_NROW = 4
_NIDX = 5


def _sc_agg_body(chunk, nch, xs_hbm, src_hbm, dst_hbm, zeros_hbm, out_hbm,
                 *refs):
    sidx = refs[0:_NIDX]
    didx = refs[_NIDX:2 * _NIDX]
    rows = refs[2 * _NIDX:2 * _NIDX + _NROW]
    acc = refs[2 * _NIDX + _NROW]
    isem = refs[2 * _NIDX + _NROW + 1:3 * _NIDX + _NROW + 1]
    gsem = refs[3 * _NIDX + _NROW + 1:3 * _NIDX + 2 * _NROW + 1]
    ssem = refs[3 * _NIDX + 2 * _NROW + 1:3 * _NIDX + 3 * _NROW + 1]

    cid = lax.axis_index("c")
    sid = lax.axis_index("s")
    base = (cid * NS + sid) * EPT

    pltpu.sync_copy(zeros_hbm, acc.at[pl.ds(sid * RPT, RPT)])
    plsc.subcore_barrier()

    def load_idx(k):
        b = k % _NIDX
        c1 = pltpu.async_copy(src_hbm.at[pl.ds(base + k * chunk, chunk)],
                              sidx[b], isem[b])
        c2 = pltpu.async_copy(dst_hbm.at[pl.ds(base + k * chunk, chunk)],
                              didx[b], isem[b])
        return (c1, c2)

    def gather(k):
        return pltpu.async_copy(xs_hbm.at[sidx[k % _NIDX]],
                                rows[k % _NROW], gsem[k % _NROW])

    def scatter(k):
        return pltpu.async_copy(rows[k % _NROW], acc.at[didx[k % _NIDX]],
                                ssem[k % _NROW], add=True)

    ng = _NROW - 1
    idx_pending = [None] * _NIDX
    for k in range(min(ng + 1, nch)):
        idx_pending[k % _NIDX] = load_idx(k)
    g = [None] * nch
    s = [None] * nch
    for k in range(min(ng, nch)):
        for c in idx_pending[k % _NIDX]:
            c.wait()
        g[k] = gather(k)

    for k in range(nch):
        g[k].wait()
        if k - 1 >= 0:
            s[k - 1].wait()
        if k + ng + 1 < nch:
            idx_pending[(k + ng + 1) % _NIDX] = load_idx(k + ng + 1)
        if k + ng < nch:
            for c in idx_pending[(k + ng) % _NIDX]:
                c.wait()
            g[k + ng] = gather(k + ng)
        s[k] = scatter(k)

    s[nch - 1].wait()

    plsc.subcore_barrier()
    pltpu.sync_copy(acc.at[pl.ds(sid * RPT, RPT)],
                    out_hbm.at[cid, pl.ds(sid * RPT, RPT)])


def _sc_agg(xs, src, dst, chunk):
    nch = EPT // chunk
    zeros = jnp.zeros((RPT, D1), jnp.float32)
    scratch = (
        [pltpu.VMEM((chunk,), jnp.int32)] * _NIDX
        + [pltpu.VMEM((chunk,), jnp.int32)] * _NIDX
        + [pltpu.VMEM((chunk, D1), jnp.float32)] * _NROW
        + [pltpu.VMEM_SHARED((NPAD, D1), jnp.float32)]
        + [pltpu.SemaphoreType.DMA] * _NIDX
        + [pltpu.SemaphoreType.DMA] * (2 * _NROW)
    )
    kern = pl.kernel(
        functools.partial(_sc_agg_body, chunk, nch),
        out_type=jax.ShapeDtypeStruct((NC, NPAD, D1), jnp.float32),
        mesh=_mesh,
        scratch_types=scratch,
    )
    return kern(xs, src, dst, zeros)


def _sc_hist_body(chunk, nch, dst_hbm, zeros_hbm, out_hbm,
                  didx0, didx1, hist, isem0, isem1):
    cid = lax.axis_index("c")
    sid = lax.axis_index("s")
    base = (cid * NS + sid) * EPT

    pltpu.sync_copy(zeros_hbm, hist)

    didx = (didx0, didx1)
    isem = (isem0, isem1)

    def load_idx(k):
        b = k % 2
        return pltpu.async_copy(dst_hbm.at[pl.ds(base + k * chunk, chunk)],
                                didx[b], isem[b])

    pending = load_idx(0)
    for k in range(nch):
        b = k % 2
        pending.wait()
        if k + 1 < nch:
            pending = load_idx(k + 1)

        @pl.loop(0, chunk, step=16)
        def _(j):
            v = didx[b][pl.ds(j, 16)]
            cnt, last = plsc.scan_count(v)
            plsc.addupdate_scatter(hist, [v], cnt, mask=last)

    pltpu.sync_copy(hist, out_hbm.at[cid * NS + sid])


def _sc_hist(dst, chunk):
    nch = EPT // chunk
    zeros = jnp.zeros((NPAD,), jnp.int32)
    cp = pltpu.CompilerParams()
    if "needs_layout_passes" in pltpu.CompilerParams.__dataclass_fields__:
        cp = dataclasses.replace(cp, needs_layout_passes=False)
    kern = pl.kernel(
        functools.partial(_sc_hist_body, chunk, nch),
        out_type=jax.ShapeDtypeStruct((NW, NPAD), jnp.int32),
        mesh=_mesh,
        compiler_params=cp,
        scratch_types=[
            pltpu.VMEM((chunk,), jnp.int32),
            pltpu.VMEM((chunk,), jnp.int32),
            pltpu.VMEM((NPAD,), jnp.int32),
            pltpu.SemaphoreType.DMA,
            pltpu.SemaphoreType.DMA,
        ],
    )
    return kern(dst, zeros)


_BD = 1024


def _tc_mm_body(x_ref, w1_ref, xw_ref):
    xw_ref[...] = jnp.dot(x_ref[...], w1_ref[...],
                          precision=lax.Precision.HIGHEST)


def _tc1_body(hist_ref, xw_ref, dinv_ref, dinv2_ref, xs_ref):
    counts = jnp.sum(hist_ref[...].astype(jnp.float32), axis=0,
                     keepdims=True)
    deg = 1.0 + jnp.transpose(counts)
    dinv = lax.rsqrt(deg)
    dinv_ref[...] = dinv
    dinv2_ref[...] = 1.0 / deg
    xs_ref[...] = xw_ref[...] * dinv


def _tc2_body(agg_ref, xw_ref, dinv_ref, dinv2_ref, b1_ref, h_ref, hs_ref):
    dinv = dinv_ref[...]
    h = jax.nn.relu((agg_ref[0] + agg_ref[1]) * dinv
                    + xw_ref[...] * dinv2_ref[...] + b1_ref[...])
    h_ref[...] = h
    hs_ref[...] = h * dinv


def _tc3_body(agg_ref, h_ref, dinv_ref, dinv2_ref, b2_ref, w2_ref, out_ref):
    ah = (agg_ref[0] + agg_ref[1]) * dinv_ref[...] \
        + h_ref[...] * dinv2_ref[...]
    out_ref[...] = jnp.dot(ah, w2_ref[...],
                           precision=lax.Precision.HIGHEST) + b2_ref[...]


_BR = 1000


def _row_spec(d):
    return pl.BlockSpec((_BR, d), lambda i: (i, 0))


def _part_spec():
    return pl.BlockSpec((NC, _BR, D1), lambda i: (0, i, 0))


def _full_spec(shape):
    nd = len(shape)
    return pl.BlockSpec(shape, lambda i: (0,) * nd)


def kernel(x, edge_index, W1, b1, W2, b2):
    src = edge_index[0]
    dst = edge_index[1]
    b1r = b1.reshape(1, D1)
    b2r = b2.reshape(1, D2)

    grid = (N // _BR,)
    dspec = _row_spec(1)

    hist = _sc_hist(dst, chunk=2000)
    xw1 = pl.pallas_call(
        _tc_mm_body,
        grid=grid,
        in_specs=[_row_spec(D1), _full_spec((D1, D1))],
        out_specs=_row_spec(D1),
        out_shape=jax.ShapeDtypeStruct((N, D1), jnp.float32),
    )(x, W1)

    dinv, dinv2, xs1 = pl.pallas_call(
        _tc1_body,
        grid=(NPAD // _BD,),
        in_specs=[pl.BlockSpec((NW, _BD), lambda i: (0, i)),
                  pl.BlockSpec((_BD, D1), lambda i: (i, 0))],
        out_specs=[pl.BlockSpec((_BD, 1), lambda i: (i, 0))] * 2
        + [pl.BlockSpec((_BD, D1), lambda i: (i, 0))],
        out_shape=[jax.ShapeDtypeStruct((NPAD, 1), jnp.float32)] * 2
        + [jax.ShapeDtypeStruct((N, D1), jnp.float32)],
    )(hist, xw1)

    agg1 = _sc_agg(xs1, src, dst, chunk=80)

    h, hs = pl.pallas_call(
        _tc2_body,
        grid=grid,
        in_specs=[_part_spec(), _row_spec(D1), dspec, dspec,
                  _full_spec((1, D1))],
        out_specs=[_row_spec(D1), _row_spec(D1)],
        out_shape=[jax.ShapeDtypeStruct((N, D1), jnp.float32)] * 2,
    )(agg1, xw1, dinv, dinv2, b1r)

    agg2 = _sc_agg(hs, src, dst, chunk=80)

    out = pl.pallas_call(
        _tc3_body,
        grid=grid,
        in_specs=[_part_spec(), _row_spec(D1), dspec, dspec,
                  _full_spec((1, D2)), _full_spec((D1, D2))],
        out_specs=_row_spec(D2),
        out_shape=jax.ShapeDtypeStruct((N, D2), jnp.float32),
    )(agg2, h, dinv, dinv2, b2r, W2)

    return out


# speedup vs baseline: 35.3123x; 1.0163x over previous
import dataclasses
import functools

import jax
import jax.numpy as jnp
from jax import lax
from jax.experimental import pallas as pl
from jax.experimental.pallas import tpu as pltpu
from jax.experimental.pallas import tpu_sc as plsc

N = 10000
E = 320000
D1 = 128
D2 = 16

NC = 2
NS = 16
NW = NC * NS
EPT = E // NW
NPAD = 10240
RPT = NPAD // NS

_mesh = plsc.VectorSubcoreMesh(core_axis_name="c", subcore_axis_name="s")


_NROW = 4
_NIDX = 5


def _sc_agg_body(chunk, nch, xs_hbm, src_hbm, dst_hbm, zeros_hbm, out_hbm,
                 *refs):
    sidx = refs[0:_NIDX]
    didx = refs[_NIDX:2 * _NIDX]
    rows = refs[2 * _NIDX:2 * _NIDX + _NROW]
    acc = refs[2 * _NIDX + _NROW]
    isem = refs[2 * _NIDX + _NROW + 1:3 * _NIDX + _NROW + 1]
    gsem = refs[3 * _NIDX + _NROW + 1:3 * _NIDX + 2 * _NROW + 1]
    ssem = refs[3 * _NIDX + 2 * _NROW + 1:3 * _NIDX + 3 * _NROW + 1]

    cid = lax.axis_index("c")
    sid = lax.axis_index("s")
    base = (cid * NS + sid) * EPT

    pltpu.sync_copy(zeros_hbm, acc.at[pl.ds(sid * RPT, RPT)])
    plsc.subcore_barrier()

    def load_idx(k):
        b = k % _NIDX
        c1 = pltpu.async_copy(src_hbm.at[pl.ds(base + k * chunk, chunk)],
                              sidx[b], isem[b])
        c2 = pltpu.async_copy(dst_hbm.at[pl.ds(base + k * chunk, chunk)],
                              didx[b], isem[b])
        return (c1, c2)

    def gather(k):
        return pltpu.async_copy(xs_hbm.at[sidx[k % _NIDX]],
                                rows[k % _NROW], gsem[k % _NROW])

    def scatter(k):
        return pltpu.async_copy(rows[k % _NROW], acc.at[didx[k % _NIDX]],
                                ssem[k % _NROW], add=True)

    ng = _NROW - 1
    idx_pending = [None] * _NIDX
    for k in range(min(ng + 1, nch)):
        idx_pending[k % _NIDX] = load_idx(k)
    g = [None] * nch
    s = [None] * nch
    for k in range(min(ng, nch)):
        for c in idx_pending[k % _NIDX]:
            c.wait()
        g[k] = gather(k)

    for k in range(nch):
        g[k].wait()
        if k - 1 >= 0:
            s[k - 1].wait()
        if k + ng + 1 < nch:
            idx_pending[(k + ng + 1) % _NIDX] = load_idx(k + ng + 1)
        if k + ng < nch:
            for c in idx_pending[(k + ng) % _NIDX]:
                c.wait()
            g[k + ng] = gather(k + ng)
        s[k] = scatter(k)

    s[nch - 1].wait()

    plsc.subcore_barrier()
    pltpu.sync_copy(acc.at[pl.ds(sid * RPT, RPT)],
                    out_hbm.at[cid, pl.ds(sid * RPT, RPT)])


def _sc_agg(xs, src, dst, chunk):
    nch = EPT // chunk
    zeros = jnp.zeros((RPT, D1), jnp.float32)
    scratch = (
        [pltpu.VMEM((chunk,), jnp.int32)] * _NIDX
        + [pltpu.VMEM((chunk,), jnp.int32)] * _NIDX
        + [pltpu.VMEM((chunk, D1), jnp.float32)] * _NROW
        + [pltpu.VMEM_SHARED((NPAD, D1), jnp.float32)]
        + [pltpu.SemaphoreType.DMA] * _NIDX
        + [pltpu.SemaphoreType.DMA] * (2 * _NROW)
    )
    kern = pl.kernel(
        functools.partial(_sc_agg_body, chunk, nch),
        out_type=jax.ShapeDtypeStruct((NC, NPAD, D1), jnp.float32),
        mesh=_mesh,
        scratch_types=scratch,
    )
    return kern(xs, src, dst, zeros)


def _sc_hist_body(chunk, nch, dst_hbm, zeros_hbm, out_hbm,
                  didx0, didx1, hist, isem0, isem1):
    cid = lax.axis_index("c")
    sid = lax.axis_index("s")
    base = (cid * NS + sid) * EPT

    pltpu.sync_copy(zeros_hbm, hist)

    didx = (didx0, didx1)
    isem = (isem0, isem1)

    def load_idx(k):
        b = k % 2
        return pltpu.async_copy(dst_hbm.at[pl.ds(base + k * chunk, chunk)],
                                didx[b], isem[b])

    pending = load_idx(0)
    for k in range(nch):
        b = k % 2
        pending.wait()
        if k + 1 < nch:
            pending = load_idx(k + 1)

        @pl.loop(0, chunk, step=16)
        def _(j):
            v = didx[b][pl.ds(j, 16)]
            cnt, last = plsc.scan_count(v)
            plsc.addupdate_scatter(hist, [v], cnt, mask=last)

    pltpu.sync_copy(hist, out_hbm.at[cid * NS + sid])


def _sc_hist(dst, chunk):
    nch = EPT // chunk
    zeros = jnp.zeros((NPAD,), jnp.int32)
    cp = pltpu.CompilerParams()
    if "needs_layout_passes" in pltpu.CompilerParams.__dataclass_fields__:
        cp = dataclasses.replace(cp, needs_layout_passes=False)
    kern = pl.kernel(
        functools.partial(_sc_hist_body, chunk, nch),
        out_type=jax.ShapeDtypeStruct((NW, NPAD), jnp.int32),
        mesh=_mesh,
        compiler_params=cp,
        scratch_types=[
            pltpu.VMEM((chunk,), jnp.int32),
            pltpu.VMEM((chunk,), jnp.int32),
            pltpu.VMEM((NPAD,), jnp.int32),
            pltpu.SemaphoreType.DMA,
            pltpu.SemaphoreType.DMA,
        ],
    )
    return kern(dst, zeros)


_BD = 1024


def _tc_mm_body(x_ref, w1_ref, xw_ref):
    xw_ref[...] = jnp.dot(x_ref[...], w1_ref[...],
                          precision=lax.Precision.HIGHEST)


def _tc1_body(hist_ref, xw_ref, dinv_ref, xs_ref):
    counts = jnp.sum(hist_ref[...].astype(jnp.float32), axis=0,
                     keepdims=True)
    dinv = lax.rsqrt(1.0 + jnp.transpose(counts))
    dinv_ref[...] = dinv
    xs_ref[...] = xw_ref[...] * dinv


def _tc2_body(agg_ref, xs_ref, dinv_ref, b1_ref, hs_ref):
    dinv = dinv_ref[...]
    h = jax.nn.relu((agg_ref[0] + agg_ref[1] + xs_ref[...]) * dinv
                    + b1_ref[...])
    hs_ref[...] = h * dinv


def _tc3_body(agg_ref, hs_ref, dinv_ref, b2_ref, w2_ref, out_ref):
    ah = (agg_ref[0] + agg_ref[1] + hs_ref[...]) * dinv_ref[...]
    out_ref[...] = jnp.dot(ah, w2_ref[...],
                           precision=lax.Precision.HIGHEST) + b2_ref[...]


_BR = 1000


def _row_spec(d):
    return pl.BlockSpec((_BR, d), lambda i: (i, 0))


def _part_spec():
    return pl.BlockSpec((NC, _BR, D1), lambda i: (0, i, 0))


def _full_spec(shape):
    nd = len(shape)
    return pl.BlockSpec(shape, lambda i: (0,) * nd)


def kernel(x, edge_index, W1, b1, W2, b2):
    src = edge_index[0]
    dst = edge_index[1]
    b1r = b1.reshape(1, D1)
    b2r = b2.reshape(1, D2)

    grid = (N // _BR,)
    dspec = _row_spec(1)

    hist = _sc_hist(dst, chunk=2000)
    xw1 = pl.pallas_call(
        _tc_mm_body,
        grid=grid,
        in_specs=[_row_spec(D1), _full_spec((D1, D1))],
        out_specs=_row_spec(D1),
        out_shape=jax.ShapeDtypeStruct((N, D1), jnp.float32),
    )(x, W1)

    dinv, xs1 = pl.pallas_call(
        _tc1_body,
        grid=(NPAD // _BD,),
        in_specs=[pl.BlockSpec((NW, _BD), lambda i: (0, i)),
                  pl.BlockSpec((_BD, D1), lambda i: (i, 0))],
        out_specs=[pl.BlockSpec((_BD, 1), lambda i: (i, 0)),
                   pl.BlockSpec((_BD, D1), lambda i: (i, 0))],
        out_shape=[jax.ShapeDtypeStruct((NPAD, 1), jnp.float32),
                   jax.ShapeDtypeStruct((N, D1), jnp.float32)],
    )(hist, xw1)

    agg1 = _sc_agg(xs1, src, dst, chunk=80)

    hs = pl.pallas_call(
        _tc2_body,
        grid=grid,
        in_specs=[_part_spec(), _row_spec(D1), dspec,
                  _full_spec((1, D1))],
        out_specs=_row_spec(D1),
        out_shape=jax.ShapeDtypeStruct((N, D1), jnp.float32),
    )(agg1, xs1, dinv, b1r)

    agg2 = _sc_agg(hs, src, dst, chunk=80)

    out = pl.pallas_call(
        _tc3_body,
        grid=grid,
        in_specs=[_part_spec(), _row_spec(D1), dspec,
                  _full_spec((1, D2)), _full_spec((D1, D2))],
        out_specs=_row_spec(D2),
        out_shape=jax.ShapeDtypeStruct((N, D2), jnp.float32),
    )(agg2, hs, dinv, b2r, W2)

    return out
